# baseline (speedup 1.0000x reference)
import functools

import jax
import jax.numpy as jnp
from jax import lax
from jax.experimental import pallas as pl
from jax.experimental.pallas import tpu as pltpu
from jax.experimental.pallas import tpu_sc as plsc

N = 10000
E = 320000
D = 128
NC = 2
NS = 16
NW = NC * NS
NP = 10240
RPT = NP // NS
EPT = E // NW
CHUNK = 128
NFULL = EPT // CHUNK
TAIL = EPT - NFULL * CHUNK
BR = 512
GRID = NP // BR

_mesh = plsc.VectorSubcoreMesh(
    core_axis_name="c", subcore_axis_name="s", num_cores=NC, num_subcores=NS)


@functools.partial(
    pl.kernel,
    out_type=jax.ShapeDtypeStruct((NW, NP), jnp.float32),
    mesh=_mesh,
    compiler_params=pltpu.CompilerParams(needs_layout_passes=False),
    scratch_types=[
        pltpu.VMEM((EPT,), jnp.int32),
        pltpu.VMEM((NP,), jnp.float32),
    ],
)
def _sc_degree(dst_hbm, zeros_hbm, out_hbm, didx, degv):
    c = lax.axis_index("c")
    s = lax.axis_index("s")
    wid = s * NC + c
    pltpu.sync_copy(zeros_hbm, degv)
    pltpu.sync_copy(dst_hbm.at[pl.ds(wid * EPT, EPT)], didx)
    ones = jnp.full((16,), 1.0, jnp.float32)

    def body(i, carry):
        idx = didx[pl.ds(i * 16, 16)]
        plsc.addupdate_scatter(degv, [idx], ones)
        return carry

    lax.fori_loop(0, EPT // 16, body, None)
    pltpu.sync_copy(degv, out_hbm.at[wid])


@functools.partial(
    pl.kernel,
    out_type=jax.ShapeDtypeStruct((NC, NP, D), jnp.float32),
    mesh=_mesh,
    scratch_types=[
        pltpu.VMEM((EPT,), jnp.int32),
        pltpu.VMEM((2, CHUNK), jnp.int32),
        pltpu.VMEM((2, CHUNK, D), jnp.float32),
        pltpu.VMEM((1, TAIL), jnp.int32),
        pltpu.VMEM((TAIL, D), jnp.float32),
        pltpu.VMEM_SHARED((NP, D), jnp.float32),
        pltpu.SemaphoreType.DMA((2,)),
        pltpu.SemaphoreType.DMA((2,)),
        pltpu.SemaphoreType.DMA((2,)),
    ],
)
def _sc_aggregate(src_hbm, dst_hbm, y_hbm, zeros_hbm, out_hbm,
                  sidx, didxr, rows, tdidx, trows, acc, gsem, dsem, ssem):
    c = lax.axis_index("c")
    s = lax.axis_index("s")
    wid = s * NC + c
    r0 = s * RPT
    e0 = wid * EPT

    @pl.when(c == 0)
    def _():
        pltpu.sync_copy(y_hbm.at[pl.ds(r0, RPT)], acc.at[pl.ds(r0, RPT)])

    @pl.when(c != 0)
    def _():
        pltpu.sync_copy(zeros_hbm.at[pl.ds(r0, RPT)], acc.at[pl.ds(r0, RPT)])

    pltpu.sync_copy(src_hbm.at[pl.ds(e0, EPT)], sidx)
    plsc.subcore_barrier()

    def fetch(j, q):
        off = j * CHUNK
        pltpu.async_copy(dst_hbm.at[pl.ds(e0 + off, CHUNK)], didxr.at[q],
                         dsem.at[q])
        pltpu.async_copy(y_hbm.at[sidx.at[pl.ds(off, CHUNK)]], rows.at[q],
                         gsem.at[q])

    fetch(0, 0)

    def body(i, carry):
        p = lax.rem(i, 2)

        @pl.when(i + 1 < NFULL)
        def _():
            @pl.when(i >= 1)
            def _():
                pltpu.make_async_copy(rows.at[1 - p], acc.at[didxr.at[1 - p]],
                                      ssem.at[1 - p]).wait()

            fetch(i + 1, 1 - p)

        pltpu.make_async_copy(dst_hbm.at[pl.ds(e0, CHUNK)], didxr.at[p],
                              dsem.at[p]).wait()
        pltpu.make_async_copy(y_hbm.at[sidx.at[pl.ds(0, CHUNK)]], rows.at[p],
                              gsem.at[p]).wait()
        pltpu.async_copy(rows.at[p], acc.at[didxr.at[p]], ssem.at[p],
                         add=True)
        return carry

    lax.fori_loop(0, NFULL, body, None)
    for k in range(NFULL - 2, NFULL):
        q = k % 2
        pltpu.make_async_copy(rows.at[q], acc.at[didxr.at[q]],
                              ssem.at[q]).wait()
    toff = e0 + NFULL * CHUNK
    pltpu.sync_copy(dst_hbm.at[pl.ds(toff, TAIL)], tdidx.at[0])
    pltpu.sync_copy(y_hbm.at[sidx.at[pl.ds(NFULL * CHUNK, TAIL)]], trows)
    pltpu.sync_copy(trows, acc.at[tdidx.at[0]], add=True)
    plsc.subcore_barrier()
    pltpu.sync_copy(acc.at[pl.ds(r0, RPT)], out_hbm.at[c, pl.ds(r0, RPT)])


def _dinv_block(dp_ref):
    deg = 1.0 + jnp.sum(dp_ref[...], axis=0)[:, None]
    return lax.rsqrt(deg)


def _prep_body(x_ref, w_ref, dp_ref, y_ref):
    dinv = _dinv_block(dp_ref)
    y_ref[...] = jnp.dot(x_ref[...], w_ref[...],
                         preferred_element_type=jnp.float32) * dinv


def _tc_prep(x, W1, degp):
    return pl.pallas_call(
        _prep_body,
        grid=(GRID,),
        in_specs=[
            pl.BlockSpec((BR, D), lambda i: (i, 0)),
            pl.BlockSpec((D, D), lambda i: (0, 0)),
            pl.BlockSpec((NW, BR), lambda i: (0, i)),
        ],
        out_specs=pl.BlockSpec((BR, D), lambda i: (i, 0)),
        out_shape=jax.ShapeDtypeStruct((NP, D), jnp.float32),
    )(x, W1, degp)


def _mid_body(z_ref, dp_ref, b_ref, w_ref, y_ref):
    dinv = _dinv_block(dp_ref)
    h = jax.nn.relu(dinv * (z_ref[0] + z_ref[1]) + b_ref[...])
    y_ref[...] = jnp.dot(h, w_ref[...],
                         preferred_element_type=jnp.float32) * dinv


def _tc_mid(z1, degp, b1, W2):
    return pl.pallas_call(
        _mid_body,
        grid=(GRID,),
        in_specs=[
            pl.BlockSpec((NC, BR, D), lambda i: (0, i, 0)),
            pl.BlockSpec((NW, BR), lambda i: (0, i)),
            pl.BlockSpec((1, D), lambda i: (0, 0)),
            pl.BlockSpec((D, D), lambda i: (0, 0)),
        ],
        out_specs=pl.BlockSpec((BR, D), lambda i: (i, 0)),
        out_shape=jax.ShapeDtypeStruct((NP, D), jnp.float32),
    )(z1, degp, b1, W2)


def _final_body(z_ref, dp_ref, b_ref, o_ref):
    dinv = _dinv_block(dp_ref)
    o_ref[...] = dinv * (z_ref[0] + z_ref[1]) + b_ref[...]


def _tc_final(z2, degp, b2):
    return pl.pallas_call(
        _final_body,
        grid=(GRID,),
        in_specs=[
            pl.BlockSpec((NC, BR, D), lambda i: (0, i, 0)),
            pl.BlockSpec((NW, BR), lambda i: (0, i)),
            pl.BlockSpec((1, D), lambda i: (0, 0)),
        ],
        out_specs=pl.BlockSpec((BR, D), lambda i: (i, 0)),
        out_shape=jax.ShapeDtypeStruct((NP, D), jnp.float32),
    )(z2, degp, b2)


def kernel(inputs, edge_index, W1, b1, W2, b2):
    src = edge_index[0].astype(jnp.int32)
    dst = edge_index[1].astype(jnp.int32)
    zerosNP = jnp.zeros((NP,), jnp.float32)
    zerosD = jnp.zeros((NP, D), jnp.float32)
    b1r = b1.reshape(1, D)
    b2r = b2.reshape(1, D)

    degp = _sc_degree(dst, zerosNP)
    y1 = _tc_prep(inputs, W1, degp)
    z1 = _sc_aggregate(src, dst, y1, zerosD)
    y2 = _tc_mid(z1, degp, b1r, W2)
    z2 = _sc_aggregate(src, dst, y2, zerosD)
    out = _tc_final(z2, degp, b2r)
    return out[:N]


# speedup vs baseline: 1.0794x; 1.0794x over previous
import functools

import jax
import jax.numpy as jnp
from jax import lax
from jax.experimental import pallas as pl
from jax.experimental.pallas import tpu as pltpu
from jax.experimental.pallas import tpu_sc as plsc

N = 10000
E = 320000
D = 128
NC = 2
NS = 16
NW = NC * NS
NP = 10240
RPT = NP // NS
EPT = E // NW
CHUNK = 80
NCHUNK = EPT // CHUNK
BR = 512
GRID = NP // BR

_mesh = plsc.VectorSubcoreMesh(
    core_axis_name="c", subcore_axis_name="s", num_cores=NC, num_subcores=NS)


@functools.partial(
    pl.kernel,
    out_type=jax.ShapeDtypeStruct((NW, NP), jnp.float32),
    mesh=_mesh,
    compiler_params=pltpu.CompilerParams(needs_layout_passes=False),
    scratch_types=[
        pltpu.VMEM((EPT,), jnp.int32),
        pltpu.VMEM((NP,), jnp.float32),
    ],
)
def _sc_degree(dst_hbm, zeros_hbm, out_hbm, didx, degv):
    c = lax.axis_index("c")
    s = lax.axis_index("s")
    wid = s * NC + c
    pltpu.sync_copy(zeros_hbm, degv)
    pltpu.sync_copy(dst_hbm.at[pl.ds(wid * EPT, EPT)], didx)
    ones = jnp.full((16,), 1.0, jnp.float32)

    def body(i, carry):
        idx = didx[pl.ds(i * 16, 16)]
        plsc.addupdate_scatter(degv, [idx], ones)
        return carry

    lax.fori_loop(0, EPT // 16, body, None)
    pltpu.sync_copy(degv, out_hbm.at[wid])


@functools.partial(
    pl.kernel,
    out_type=jax.ShapeDtypeStruct((NC, NP, D), jnp.float32),
    mesh=_mesh,
    scratch_types=[
        pltpu.VMEM((EPT,), jnp.int32),
        pltpu.VMEM((3, CHUNK), jnp.int32),
        pltpu.VMEM((3, CHUNK, D), jnp.float32),
        pltpu.VMEM_SHARED((NP, D), jnp.float32),
        pltpu.SemaphoreType.DMA((3,)),
        pltpu.SemaphoreType.DMA((3,)),
        pltpu.SemaphoreType.DMA((3,)),
    ],
)
def _sc_aggregate(src_hbm, dst_hbm, y_hbm, zeros_hbm, out_hbm,
                  sidx, didxr, rows, acc, gsem, dsem, ssem):
    c = lax.axis_index("c")
    s = lax.axis_index("s")
    wid = s * NC + c
    r0 = s * RPT
    e0 = wid * EPT

    @pl.when(c == 0)
    def _():
        pltpu.sync_copy(y_hbm.at[pl.ds(r0, RPT)], acc.at[pl.ds(r0, RPT)])

    @pl.when(c != 0)
    def _():
        pltpu.sync_copy(zeros_hbm.at[pl.ds(r0, RPT)], acc.at[pl.ds(r0, RPT)])

    pltpu.sync_copy(src_hbm.at[pl.ds(e0, EPT)], sidx)
    plsc.subcore_barrier()

    def fetch(j, q):
        off = j * CHUNK
        pltpu.async_copy(dst_hbm.at[pl.ds(e0 + off, CHUNK)], didxr.at[q],
                         dsem.at[q])
        pltpu.async_copy(y_hbm.at[sidx.at[pl.ds(off, CHUNK)]], rows.at[q],
                         gsem.at[q])

    fetch(0, 0)
    fetch(1, 1)

    def body(i, carry):
        p = lax.rem(i, 3)

        @pl.when(i + 2 < NCHUNK)
        def _():
            q = lax.rem(i + 2, 3)

            @pl.when(i >= 1)
            def _():
                pltpu.make_async_copy(rows.at[q], acc.at[didxr.at[q]],
                                      ssem.at[q]).wait()

            fetch(i + 2, q)

        pltpu.make_async_copy(dst_hbm.at[pl.ds(e0, CHUNK)], didxr.at[p],
                              dsem.at[p]).wait()
        pltpu.make_async_copy(y_hbm.at[sidx.at[pl.ds(0, CHUNK)]], rows.at[p],
                              gsem.at[p]).wait()
        pltpu.async_copy(rows.at[p], acc.at[didxr.at[p]], ssem.at[p],
                         add=True)
        return carry

    lax.fori_loop(0, NCHUNK, body, None)
    for k in range(NCHUNK - 3, NCHUNK):
        q = k % 3
        pltpu.make_async_copy(rows.at[q], acc.at[didxr.at[q]],
                              ssem.at[q]).wait()
    plsc.subcore_barrier()
    pltpu.sync_copy(acc.at[pl.ds(r0, RPT)], out_hbm.at[c, pl.ds(r0, RPT)])


def _dinv_block(dp_ref):
    deg = 1.0 + jnp.sum(dp_ref[...], axis=0)[:, None]
    return lax.rsqrt(deg)


def _prep_body(x_ref, w_ref, dp_ref, y_ref):
    dinv = _dinv_block(dp_ref)
    y_ref[...] = jnp.dot(x_ref[...], w_ref[...],
                         preferred_element_type=jnp.float32) * dinv


def _tc_prep(x, W1, degp):
    return pl.pallas_call(
        _prep_body,
        grid=(GRID,),
        in_specs=[
            pl.BlockSpec((BR, D), lambda i: (i, 0)),
            pl.BlockSpec((D, D), lambda i: (0, 0)),
            pl.BlockSpec((NW, BR), lambda i: (0, i)),
        ],
        out_specs=pl.BlockSpec((BR, D), lambda i: (i, 0)),
        out_shape=jax.ShapeDtypeStruct((NP, D), jnp.float32),
    )(x, W1, degp)


def _mid_body(z_ref, dp_ref, b_ref, w_ref, y_ref):
    dinv = _dinv_block(dp_ref)
    h = jax.nn.relu(dinv * (z_ref[0] + z_ref[1]) + b_ref[...])
    y_ref[...] = jnp.dot(h, w_ref[...],
                         preferred_element_type=jnp.float32) * dinv


def _tc_mid(z1, degp, b1, W2):
    return pl.pallas_call(
        _mid_body,
        grid=(GRID,),
        in_specs=[
            pl.BlockSpec((NC, BR, D), lambda i: (0, i, 0)),
            pl.BlockSpec((NW, BR), lambda i: (0, i)),
            pl.BlockSpec((1, D), lambda i: (0, 0)),
            pl.BlockSpec((D, D), lambda i: (0, 0)),
        ],
        out_specs=pl.BlockSpec((BR, D), lambda i: (i, 0)),
        out_shape=jax.ShapeDtypeStruct((NP, D), jnp.float32),
    )(z1, degp, b1, W2)


def _final_body(z_ref, dp_ref, b_ref, o_ref):
    dinv = _dinv_block(dp_ref)
    o_ref[...] = dinv * (z_ref[0] + z_ref[1]) + b_ref[...]


def _tc_final(z2, degp, b2):
    return pl.pallas_call(
        _final_body,
        grid=(GRID,),
        in_specs=[
            pl.BlockSpec((NC, BR, D), lambda i: (0, i, 0)),
            pl.BlockSpec((NW, BR), lambda i: (0, i)),
            pl.BlockSpec((1, D), lambda i: (0, 0)),
        ],
        out_specs=pl.BlockSpec((BR, D), lambda i: (i, 0)),
        out_shape=jax.ShapeDtypeStruct((NP, D), jnp.float32),
    )(z2, degp, b2)


def kernel(inputs, edge_index, W1, b1, W2, b2):
    src = edge_index[0].astype(jnp.int32)
    dst = edge_index[1].astype(jnp.int32)
    zerosNP = jnp.zeros((NP,), jnp.float32)
    zerosD = jnp.zeros((NP, D), jnp.float32)
    b1r = b1.reshape(1, D)
    b2r = b2.reshape(1, D)

    degp = _sc_degree(dst, zerosNP)
    y1 = _tc_prep(inputs, W1, degp)
    z1 = _sc_aggregate(src, dst, y1, zerosD)
    y2 = _tc_mid(z1, degp, b1r, W2)
    z2 = _sc_aggregate(src, dst, y2, zerosD)
    out = _tc_final(z2, degp, b2r)
    return out[:N]
